# Initial kernel scaffold; baseline (speedup 1.0000x reference)
import jax
import jax.numpy as jnp
from jax.experimental import pallas as pl


def kernel(x, W):
    raise NotImplementedError("write your pallas kernel here")


# speedup vs baseline: 5.1580x; 5.1580x over previous
import functools

import jax
import jax.numpy as jnp
from jax import lax
from jax.experimental import pallas as pl
from jax.experimental.pallas import tpu as pltpu
from jax.experimental.pallas import tpu_sc as plsc

NUM_CORES = 2
NUM_SUBCORES = 16
NUM_WORKERS = NUM_CORES * NUM_SUBCORES
CHUNK = 128


@functools.partial(jax.jit, static_argnums=(2, 3))
def _embedding_gather(x_flat, W, B, D):
  b_per_w = B // NUM_WORKERS
  n_chunks = b_per_w // CHUNK
  mesh = plsc.VectorSubcoreMesh(
      core_axis_name="c", subcore_axis_name="s",
      num_cores=NUM_CORES, num_subcores=NUM_SUBCORES)

  @functools.partial(
      pl.kernel,
      out_type=jax.ShapeDtypeStruct((B, D), jnp.float32),
      mesh=mesh,
      scratch_types=[
          pltpu.VMEM((CHUNK,), jnp.int32),
          pltpu.VMEM((CHUNK, D), jnp.float32),
          pltpu.SemaphoreType.DMA,
      ],
  )
  def k(table_hbm, idx_hbm, out_hbm, idx_v, rows_v, sem):
    wid = lax.axis_index("s") * NUM_CORES + lax.axis_index("c")
    base = wid * b_per_w

    def body(i, carry):
      off = base + i * CHUNK
      pltpu.sync_copy(idx_hbm.at[pl.ds(off, CHUNK)], idx_v)
      pltpu.async_copy(table_hbm.at[idx_v], rows_v, sem).wait()
      pltpu.sync_copy(rows_v, out_hbm.at[pl.ds(off, CHUNK)])
      return carry

    lax.fori_loop(0, n_chunks, body, 0)

  return k(W, x_flat)


def kernel(x, W):
  batch, seq = x.shape
  D = W.shape[-1]
  B = batch * seq
  x_flat = x.reshape(B).astype(jnp.int32)
  out = _embedding_gather(x_flat, W, B, D)
  return out.reshape(batch, seq, D)


# speedup vs baseline: 9.0892x; 1.7622x over previous
import functools

import jax
import jax.numpy as jnp
from jax import lax
from jax.experimental import pallas as pl
from jax.experimental.pallas import tpu as pltpu
from jax.experimental.pallas import tpu_sc as plsc

NUM_CORES = 2
NUM_SUBCORES = 16
NUM_WORKERS = NUM_CORES * NUM_SUBCORES
CHUNK = 128
NBUF = 4


@functools.partial(jax.jit, static_argnums=(2, 3))
def _embedding_gather(x_flat, W, B, D):
  b_per_w = B // NUM_WORKERS
  n_chunks = b_per_w // CHUNK
  n_groups = n_chunks // NBUF
  mesh = plsc.VectorSubcoreMesh(
      core_axis_name="c", subcore_axis_name="s",
      num_cores=NUM_CORES, num_subcores=NUM_SUBCORES)

  @functools.partial(
      pl.kernel,
      out_type=jax.ShapeDtypeStruct((B, D), jnp.float32),
      mesh=mesh,
      scratch_types=(
          [pltpu.VMEM((b_per_w,), jnp.int32)]
          + [pltpu.VMEM((CHUNK, D), jnp.float32) for _ in range(NBUF)]
          + [pltpu.SemaphoreType.DMA for _ in range(2 * NBUF)]
      ),
  )
  def k(table_hbm, idx_hbm, out_hbm, idx_all, *bufs_and_sems):
    rows = bufs_and_sems[:NBUF]
    sg = bufs_and_sems[NBUF:2 * NBUF]
    sw = bufs_and_sems[2 * NBUF:3 * NBUF]
    wid = lax.axis_index("s") * NUM_CORES + lax.axis_index("c")
    base = wid * b_per_w

    pltpu.sync_copy(idx_hbm.at[pl.ds(base, b_per_w)], idx_all)

    def start_gather(i, b):
      pltpu.async_copy(
          table_hbm.at[idx_all.at[pl.ds(i * CHUNK, CHUNK)]], rows[b], sg[b])

    def wait_gather(b):
      pltpu.make_async_copy(
          table_hbm.at[idx_all.at[pl.ds(0, CHUNK)]], rows[b], sg[b]).wait()

    def start_wb(i, b):
      pltpu.async_copy(rows[b], out_hbm.at[pl.ds(base + i * CHUNK, CHUNK)],
                       sw[b])

    def wait_wb(b):
      pltpu.make_async_copy(rows[b], out_hbm.at[pl.ds(base, CHUNK)],
                            sw[b]).wait()

    for b in range(NBUF):
      start_gather(b, b)

    def group(g, carry):
      for b in range(NBUF):
        wait_gather(b)
        start_wb(g * NBUF + b, b)
      for b in range(NBUF):
        @pl.when(g + 1 < n_groups)
        def _():
          wait_wb(b)
          start_gather((g + 1) * NBUF + b, b)
      return carry

    lax.fori_loop(0, n_groups, group, 0)
    for b in range(NBUF):
      wait_wb(b)

  return k(W, x_flat)


def kernel(x, W):
  batch, seq = x.shape
  D = W.shape[-1]
  B = batch * seq
  x_flat = x.reshape(B).astype(jnp.int32)
  out = _embedding_gather(x_flat, W, B, D)
  return out.reshape(batch, seq, D)
